# baseline (speedup 1.0000x reference)
import jax
import jax.numpy as jnp
from jax import lax
from jax.experimental import pallas as pl
from jax.experimental.pallas import tpu as pltpu
from jax.experimental.pallas import tpu_sc as plsc

DEPTH = 1000
ROWS = 4096 * 26
NUM_CORES = 2
NUM_SUBCORES = 16
NW = NUM_CORES * NUM_SUBCORES
ROWS_PER_W = ROWS // NW
BLK_ROWS = 32
NBLK = ROWS_PER_W // BLK_ROWS
BLK_WORDS = BLK_ROWS * DEPTH
LANE = 16


def _onehot_sc_body(idx_hbm, onoff_hbm, out_hbm,
                    idx_v, onoff_v, buf0, buf1, sem0, sem1):
    wid = lax.axis_index("s") * NUM_CORES + lax.axis_index("c")
    row0 = wid * ROWS_PER_W
    out_base = row0 * DEPTH

    pltpu.sync_copy(idx_hbm.at[pl.ds(row0, ROWS_PER_W)], idx_v)
    pltpu.sync_copy(onoff_hbm, onoff_v)
    on_vec = onoff_v[pl.ds(0, LANE)]
    off_vec = onoff_v[pl.ds(LANE, LANE)]
    lane = lax.iota(jnp.int32, LANE)

    bufs = (buf0, buf1)
    sems = (sem0, sem1)

    def fill(buf):
        def body(i, c):
            buf[pl.ds(i * LANE, LANE)] = off_vec
            return c
        lax.fori_loop(0, BLK_WORDS // LANE, body, 0)

    fill(buf0)
    fill(buf1)

    def scat_block(buf, g, val):
        for j in range(BLK_ROWS // LANE):
            idx16 = idx_v[pl.ds(g * BLK_ROWS + j * LANE, LANE)]
            scat = (lane + j * LANE) * DEPTH + idx16
            plsc.store_scatter(buf, [scat], val)

    def start_dma(buf, sem, g):
        pltpu.async_copy(
            buf, out_hbm.at[pl.ds(out_base + g * BLK_WORDS, BLK_WORDS)], sem)

    def wait_dma(buf, sem, g):
        pltpu.make_async_copy(
            buf, out_hbm.at[pl.ds(out_base + g * BLK_WORDS, BLK_WORDS)],
            sem).wait()

    for b in range(2):
        scat_block(bufs[b], b, on_vec)
        start_dma(bufs[b], sems[b], b)

    def body(i, c):
        for b in range(2):
            g = 2 * i + b
            wait_dma(bufs[b], sems[b], g - 2)
            scat_block(bufs[b], g - 2, off_vec)
            scat_block(bufs[b], g, on_vec)
            start_dma(bufs[b], sems[b], g)
        return c

    lax.fori_loop(1, NBLK // 2, body, 0)

    for b in range(2):
        wait_dma(bufs[b], sems[b], NBLK - 2 + b)


def kernel(indices, on_value, off_value):
    idx = indices.reshape(-1).astype(jnp.int32)
    onoff = jnp.concatenate([
        jnp.full((LANE,), on_value, jnp.float32),
        jnp.full((LANE,), off_value, jnp.float32),
    ])
    mesh = plsc.VectorSubcoreMesh(
        core_axis_name="c", subcore_axis_name="s",
        num_cores=NUM_CORES, num_subcores=NUM_SUBCORES)
    out = pl.kernel(
        _onehot_sc_body,
        out_type=jax.ShapeDtypeStruct((ROWS * DEPTH,), jnp.float32),
        mesh=mesh,
        compiler_params=pltpu.CompilerParams(needs_layout_passes=False),
        scratch_types=[
            pltpu.VMEM((ROWS_PER_W,), jnp.int32),
            pltpu.VMEM((2 * LANE,), jnp.float32),
            pltpu.VMEM((BLK_WORDS,), jnp.float32),
            pltpu.VMEM((BLK_WORDS,), jnp.float32),
            pltpu.SemaphoreType.DMA,
            pltpu.SemaphoreType.DMA,
        ],
    )(idx, onoff)
    return out.reshape(indices.shape + (DEPTH,))
